# baseline (speedup 1.0000x reference)
import functools

import jax
import jax.numpy as jnp
from jax import lax
from jax.experimental import pallas as pl
from jax.experimental.pallas import tpu as pltpu
from jax.experimental.pallas import tpu_sc as plsc

N, F, E = 10000, 128, 320000
B, A, H = 256, 10, 128

NS, L = 16, 16
HPT = 320
NHALF = 2
RPT = HPT * NHALF
NPAD = NS * RPT
BACC = B + 16
CHUNK = 64
NCH = HPT // CHUNK
GPT = B // NS
CUR_PT = B // NS


def _elu(x):
    return jnp.where(x > 0, x, jnp.exp(x) - 1.0)


def _encoder_kernel(nf_ref, w1_ref, b1_ref, w2_ref, b2_ref, w3_ref, b3_ref,
                    h_ref):
    f32 = jnp.float32
    x = nf_ref[...]
    h = _elu(jnp.dot(x, w1_ref[...], preferred_element_type=f32) + b1_ref[...])
    h = _elu(jnp.dot(h, w2_ref[...], preferred_element_type=f32) + b2_ref[...])
    h = _elu(jnp.dot(h, w3_ref[...], preferred_element_type=f32) + b3_ref[...])
    h_ref[...] = h


def _sc_pool_body(h_hbm, batch3d_hbm, batch1d_hbm, cur_hbm,
                  out_sum, out_max, out_cur,
                  rows_v, idx_v, idx_flat, acc_max,
                  cur_idx, cur_rows, mbuf, macc,
                  shared_sum, shared_max, sem):
    s = lax.axis_index("s")
    row0 = s * RPT
    n = jnp.clip(N - row0, 0, RPT)

    neg16 = jnp.full((L,), -jnp.inf, dtype=jnp.float32)
    zero16 = jnp.zeros((L,), dtype=jnp.float32)

    def init_acc(i, carry):
        for j in range(F // L):
            acc_max[i, pl.ds(L * j, L)] = neg16
        return carry
    lax.fori_loop(0, BACC, init_acc, 0)

    def init_z(i, carry):
        for j in range(F // L):
            mbuf[i, pl.ds(L * j, L)] = zero16
        return carry
    lax.fori_loop(0, GPT, init_z, 0)

    with jax.named_scope("zero_shared"):
        pltpu.sync_copy(mbuf, shared_sum.at[pl.ds(GPT * s, GPT)])

        @pl.when(s == 0)
        def _zero_pad_rows():
            pltpu.sync_copy(mbuf, shared_sum.at[pl.ds(B, BACC - B)])

    plsc.subcore_barrier()

    state = (jnp.int32(-1), tuple(neg16 for _ in range(F // L)))
    for hh in range(NHALF):
        h0 = row0 + HPT * hh
        nh = jnp.clip(n - HPT * hh, 0, HPT)

        with jax.named_scope("stage"):
            pltpu.sync_copy(h_hbm.at[pl.ds(h0, HPT)], rows_v)
            pltpu.sync_copy(batch3d_hbm.at[s * NHALF + hh], idx_v)
            pltpu.sync_copy(batch1d_hbm.at[pl.ds(h0, HPT)], idx_flat)

        with jax.named_scope("scatter_add"):
            for k in range(NCH):
                pltpu.sync_copy(rows_v.at[pl.ds(CHUNK * k, CHUNK)],
                                shared_sum.at[idx_v.at[k]], add=True)

        def max_group(g, state):
            prev, regs = state
            bvec = idx_flat[pl.ds(L * g, L)]
            for lane in range(L):
                b = bvec[lane]
                r = L * g + lane
                pen = jnp.where(b != prev, -jnp.inf, 0.0).astype(jnp.float32)
                pen_vec = jnp.broadcast_to(pen, (L,))
                new_regs = []
                for j in range(F // L):
                    row = rows_v[r, pl.ds(L * j, L)]
                    v = jnp.maximum(regs[j] + pen_vec, row)
                    acc_max[b, pl.ds(L * j, L)] = v
                    new_regs.append(v)
                regs = tuple(new_regs)
                prev = b
            return (prev, regs)

        with jax.named_scope("max_loop"):
            state = lax.fori_loop(0, nh // L, max_group, state)

    with jax.named_scope("publish_max"):
        pltpu.sync_copy(acc_max.at[pl.ds(0, B)], shared_max.at[s])

    with jax.named_scope("cur_gather"):
        pltpu.sync_copy(cur_hbm.at[pl.ds(s * CUR_PT, CUR_PT)], cur_idx)
        pltpu.async_copy(h_hbm.at[cur_idx], cur_rows, sem).wait()
        pltpu.sync_copy(cur_rows, out_cur.at[pl.ds(s * CUR_PT, CUR_PT)])

    plsc.subcore_barrier()

    seg0 = GPT * s
    with jax.named_scope("merge"):
        pltpu.sync_copy(shared_max.at[0, pl.ds(seg0, GPT)], macc)
        for t in range(1, NS):
            pltpu.sync_copy(shared_max.at[t, pl.ds(seg0, GPT)], mbuf)

            def merge_body(i, carry):
                for j in range(F // L):
                    sl = pl.ds(L * j, L)
                    macc[i, sl] = jnp.maximum(macc[i, sl], mbuf[i, sl])
                return carry
            lax.fori_loop(0, GPT, merge_body, 0)

    with jax.named_scope("writeout"):
        pltpu.sync_copy(macc, out_max.at[pl.ds(seg0, GPT)])
        pltpu.sync_copy(shared_sum.at[pl.ds(seg0, GPT)],
                        out_sum.at[pl.ds(seg0, GPT)])


_sc_pool = functools.partial(
    pl.kernel,
    out_type=(
        jax.ShapeDtypeStruct((B, F), jnp.float32),
        jax.ShapeDtypeStruct((B, F), jnp.float32),
        jax.ShapeDtypeStruct((B, F), jnp.float32),
    ),
    mesh=plsc.VectorSubcoreMesh(core_axis_name="c", subcore_axis_name="s",
                                num_cores=1, num_subcores=NS),
    scratch_types=[
        pltpu.VMEM((HPT, F), jnp.float32),
        pltpu.VMEM((NCH, CHUNK), jnp.int32),
        pltpu.VMEM((HPT,), jnp.int32),
        pltpu.VMEM((BACC, F), jnp.float32),
        pltpu.VMEM((CUR_PT,), jnp.int32),
        pltpu.VMEM((CUR_PT, F), jnp.float32),
        pltpu.VMEM((GPT, F), jnp.float32),
        pltpu.VMEM((GPT, F), jnp.float32),
        pltpu.VMEM_SHARED((BACC, F), jnp.float32),
        pltpu.VMEM_SHARED((NS, B, F), jnp.float32),
        pltpu.SemaphoreType.DMA,
    ],
)(_sc_pool_body)


def _heads_kernel(sum_ref, max_ref, batch_ref, cur_ref, mask_ref,
                  wa1_ref, ba1_ref, wa2_ref, ba2_ref,
                  wc1_ref, bc1_ref, wc2_ref, bc2_ref,
                  probs_ref, values_ref):
    f32 = jnp.float32
    seg_sum = sum_ref[...]
    max_p = max_ref[...]
    CC = 2000
    seg_iota = lax.broadcasted_iota(jnp.int32, (B, CC), 0)
    counts = jnp.zeros((B, 1), dtype=f32)
    for off in range(0, N, CC):
        onehot = (seg_iota == batch_ref[:, off:off + CC]).astype(f32)
        counts = counts + jnp.sum(onehot, axis=1, keepdims=True)
    mean_p = seg_sum / jnp.maximum(counts, 1.0)
    max_p = jnp.where(counts > 0, max_p, -jnp.inf)

    graph_emb = jnp.concatenate([mean_p, max_p], axis=-1)
    actor_in = jnp.concatenate([graph_emb, cur_ref[...]], axis=-1)
    a = _elu(jnp.dot(actor_in, wa1_ref[...], preferred_element_type=f32)
             + ba1_ref[...])
    logits = jnp.dot(a, wa2_ref[...], preferred_element_type=f32) + ba2_ref[...]

    amask = mask_ref[...]
    has_valid = jnp.sum(amask, axis=-1, keepdims=True) > 0
    safe_mask = jnp.where(has_valid, amask, jnp.ones_like(amask))
    logits = jnp.where(safe_mask == 0, -jnp.inf, logits)
    m = jnp.max(logits, axis=-1, keepdims=True)
    e = jnp.exp(logits - m)
    probs = e / jnp.sum(e, axis=-1, keepdims=True)
    nan_mask = jnp.any(jnp.isnan(probs), axis=-1, keepdims=True)
    probs_ref[...] = jnp.where(nan_mask, jnp.full_like(probs, 1.0 / A), probs)

    c = _elu(jnp.dot(graph_emb, wc1_ref[...], preferred_element_type=f32)
             + bc1_ref[...])
    values_ref[...] = (jnp.dot(c, wc2_ref[...], preferred_element_type=f32)
                       + bc2_ref[...])


@jax.jit
def _run(node_features, action_mask, current_node, batch,
         W1, b1, W2, b2, W3, b3, Wa1, ba1, Wa2, ba2, Wc1, bc1, Wc2, bc2):
    f32 = jnp.float32
    nf_pad = jnp.zeros((NPAD, F), dtype=f32).at[:N].set(node_features)
    batch_pad = jnp.full((NPAD,), B, dtype=jnp.int32).at[:N].set(
        batch.astype(jnp.int32))
    batch3d = batch_pad.reshape(NS * NHALF, NCH, CHUNK)
    cur = current_node.astype(jnp.int32)

    h = pl.pallas_call(
        _encoder_kernel,
        out_shape=jax.ShapeDtypeStruct((NPAD, F), f32),
    )(nf_pad, W1, b1.reshape(1, H), W2, b2.reshape(1, H), W3, b3.reshape(1, H))

    seg_sum, seg_max, cur_emb = _sc_pool(h, batch3d, batch_pad, cur)

    return pl.pallas_call(
        _heads_kernel,
        out_shape=(jax.ShapeDtypeStruct((B, A), f32),
                   jax.ShapeDtypeStruct((B, 1), f32)),
    )(seg_sum, seg_max, batch.astype(jnp.int32).reshape(1, N), cur_emb,
      action_mask,
      Wa1, ba1.reshape(1, 256), Wa2, ba2.reshape(1, A),
      Wc1, bc1.reshape(1, 256), Wc2, bc2.reshape(1, 1))


def kernel(node_features, edge_index, edge_features, action_mask, current_node,
           batch, W1, b1, W2, b2, W3, b3, Wa1, ba1, Wa2, ba2, Wc1, bc1,
           Wc2, bc2):
    del edge_index, edge_features
    return _run(node_features, action_mask, current_node, batch,
                W1, b1, W2, b2, W3, b3, Wa1, ba1, Wa2, ba2,
                Wc1, bc1, Wc2, bc2)


# speedup vs baseline: 1.0917x; 1.0917x over previous
import functools

import jax
import jax.numpy as jnp
from jax import lax
from jax.experimental import pallas as pl
from jax.experimental.pallas import tpu as pltpu
from jax.experimental.pallas import tpu_sc as plsc

N, F, E = 10000, 128, 320000
B, A, H = 256, 10, 128

NC, NS, L = 2, 16, 16
RPT = 320
NPAD = NC * NS * RPT
BACC = B + 16
CHUNK = 64
NCH = RPT // CHUNK
GPT = B // NS
CUR_PT = B // (NC * NS)


def _elu(x):
    return jnp.where(x > 0, x, jnp.exp(x) - 1.0)


def _encoder_kernel(nf_ref, w1_ref, b1_ref, w2_ref, b2_ref, w3_ref, b3_ref,
                    h_ref):
    f32 = jnp.float32
    x = nf_ref[...]
    h = _elu(jnp.dot(x, w1_ref[...], preferred_element_type=f32) + b1_ref[...])
    h = _elu(jnp.dot(h, w2_ref[...], preferred_element_type=f32) + b2_ref[...])
    h = _elu(jnp.dot(h, w3_ref[...], preferred_element_type=f32) + b3_ref[...])
    h_ref[...] = h


def _sc_pool_body(h_hbm, batch3d_hbm, cur_hbm,
                  out_sum, out_cur,
                  rows_v, idx_v, cur_idx, cur_rows, mbuf,
                  shared_sum, sem, sem2):
    c = lax.axis_index("c")
    s = lax.axis_index("s")
    wid = c * NS + s
    row0 = wid * RPT

    zero16 = jnp.zeros((L,), dtype=jnp.float32)

    rows_cp = pltpu.make_async_copy(h_hbm.at[pl.ds(row0, RPT)], rows_v, sem)
    rows_cp.start()
    idx_cp = pltpu.make_async_copy(batch3d_hbm.at[wid], idx_v, sem2)
    idx_cp.start()

    def init_z(i, carry):
        for j in range(F // L):
            mbuf[i, pl.ds(L * j, L)] = zero16
        return carry
    lax.fori_loop(0, GPT, init_z, 0)

    with jax.named_scope("zero_shared"):
        pltpu.sync_copy(mbuf, shared_sum.at[pl.ds(GPT * s, GPT)])

        @pl.when(s == 0)
        def _zero_pad_rows():
            pltpu.sync_copy(mbuf, shared_sum.at[pl.ds(B, BACC - B)])

    rows_cp.wait()
    idx_cp.wait()
    plsc.subcore_barrier()

    with jax.named_scope("scatter_add"):
        descs = []
        for k in range(NCH):
            descs.append(pltpu.async_copy(
                rows_v.at[pl.ds(CHUNK * k, CHUNK)],
                shared_sum.at[idx_v.at[k]], sem, add=True))

    with jax.named_scope("cur_gather"):
        pltpu.sync_copy(cur_hbm.at[pl.ds(wid * CUR_PT, CUR_PT)], cur_idx)
        pltpu.async_copy(h_hbm.at[cur_idx], cur_rows, sem2).wait()
        pltpu.sync_copy(cur_rows, out_cur.at[pl.ds(wid * CUR_PT, CUR_PT)])

    with jax.named_scope("drain"):
        for d in descs:
            d.wait()

    plsc.subcore_barrier()

    with jax.named_scope("writeout"):
        pltpu.sync_copy(shared_sum.at[pl.ds(GPT * s, GPT)],
                        out_sum.at[c, pl.ds(GPT * s, GPT)])


_sc_pool = functools.partial(
    pl.kernel,
    out_type=(
        jax.ShapeDtypeStruct((NC, B, F), jnp.float32),
        jax.ShapeDtypeStruct((B, F), jnp.float32),
    ),
    mesh=plsc.VectorSubcoreMesh(core_axis_name="c", subcore_axis_name="s",
                                num_cores=NC, num_subcores=NS),
    scratch_types=[
        pltpu.VMEM((RPT, F), jnp.float32),
        pltpu.VMEM((NCH, CHUNK), jnp.int32),
        pltpu.VMEM((CUR_PT,), jnp.int32),
        pltpu.VMEM((CUR_PT, F), jnp.float32),
        pltpu.VMEM((GPT, F), jnp.float32),
        pltpu.VMEM_SHARED((BACC, F), jnp.float32),
        pltpu.SemaphoreType.DMA,
        pltpu.SemaphoreType.DMA,
    ],
)(_sc_pool_body)


def _scan_kernel(h_ref, bcol_ref, brow_ref, maxp_ref, cnt_ref):
    f32 = jnp.float32
    h = h_ref[...][:N]
    batch_row = brow_ref[...]

    h_scan = h
    b_col = bcol_ref[...]
    step = 1
    while step < N:
        shifted = jnp.concatenate(
            [jnp.full((step, F), -jnp.inf, dtype=f32),
             h_scan[:N - step, :]], axis=0)
        bshift = jnp.concatenate(
            [jnp.full((step, 1), -1, dtype=jnp.int32),
             b_col[:N - step, :]], axis=0)
        same = (b_col == bshift)
        h_scan = jnp.where(same, jnp.maximum(h_scan, shifted), h_scan)
        step *= 2

    nxt = jnp.concatenate(
        [batch_row[:, 1:], jnp.full((1, 1), -1, dtype=jnp.int32)], axis=1)
    is_last = (batch_row != nxt).astype(f32)

    C = 2000
    seg_iota = lax.broadcasted_iota(jnp.int32, (B, C), 0)
    max_p = jnp.zeros((B, F), dtype=f32)
    counts = jnp.zeros((B, 1), dtype=f32)
    for off in range(0, N, C):
        onehot = (seg_iota == batch_row[:, off:off + C]).astype(f32)
        counts = counts + jnp.sum(onehot, axis=1, keepdims=True)
        lastsel = onehot * is_last[:, off:off + C]
        max_p = max_p + jnp.dot(lastsel, h_scan[off:off + C, :],
                                preferred_element_type=f32)
    maxp_ref[...] = jnp.where(counts > 0, max_p, -jnp.inf)
    cnt_ref[...] = counts


def _heads_kernel(sum_ref, maxp_ref, cnt_ref, cur_ref, mask_ref,
                  wa1_ref, ba1_ref, wa2_ref, ba2_ref,
                  wc1_ref, bc1_ref, wc2_ref, bc2_ref,
                  probs_ref, values_ref):
    f32 = jnp.float32
    seg_sum = sum_ref[0] + sum_ref[1]
    counts = cnt_ref[...]
    mean_p = seg_sum / jnp.maximum(counts, 1.0)

    graph_emb = jnp.concatenate([mean_p, maxp_ref[...]], axis=-1)
    actor_in = jnp.concatenate([graph_emb, cur_ref[...]], axis=-1)
    a = _elu(jnp.dot(actor_in, wa1_ref[...], preferred_element_type=f32)
             + ba1_ref[...])
    logits = jnp.dot(a, wa2_ref[...], preferred_element_type=f32) + ba2_ref[...]

    amask = mask_ref[...]
    has_valid = jnp.sum(amask, axis=-1, keepdims=True) > 0
    safe_mask = jnp.where(has_valid, amask, jnp.ones_like(amask))
    logits = jnp.where(safe_mask == 0, -jnp.inf, logits)
    m = jnp.max(logits, axis=-1, keepdims=True)
    e = jnp.exp(logits - m)
    probs = e / jnp.sum(e, axis=-1, keepdims=True)
    nan_mask = jnp.any(jnp.isnan(probs), axis=-1, keepdims=True)
    probs_ref[...] = jnp.where(nan_mask, jnp.full_like(probs, 1.0 / A), probs)

    c = _elu(jnp.dot(graph_emb, wc1_ref[...], preferred_element_type=f32)
             + bc1_ref[...])
    values_ref[...] = (jnp.dot(c, wc2_ref[...], preferred_element_type=f32)
                       + bc2_ref[...])


@jax.jit
def _run(node_features, action_mask, current_node, batch,
         W1, b1, W2, b2, W3, b3, Wa1, ba1, Wa2, ba2, Wc1, bc1, Wc2, bc2):
    f32 = jnp.float32
    nf_pad = jnp.zeros((NPAD, F), dtype=f32).at[:N].set(node_features)
    batch_i32 = batch.astype(jnp.int32)
    batch_pad = jnp.full((NPAD,), B, dtype=jnp.int32).at[:N].set(batch_i32)
    batch3d = batch_pad.reshape(NC * NS, NCH, CHUNK)
    cur = current_node.astype(jnp.int32)

    h = pl.pallas_call(
        _encoder_kernel,
        out_shape=jax.ShapeDtypeStruct((NPAD, F), f32),
    )(nf_pad, W1, b1.reshape(1, H), W2, b2.reshape(1, H), W3, b3.reshape(1, H))

    seg_sum, cur_emb = _sc_pool(h, batch3d, cur)

    max_p, counts = pl.pallas_call(
        _scan_kernel,
        out_shape=(jax.ShapeDtypeStruct((B, F), f32),
                   jax.ShapeDtypeStruct((B, 1), f32)),
    )(h, batch_i32.reshape(N, 1), batch_i32.reshape(1, N))

    return pl.pallas_call(
        _heads_kernel,
        out_shape=(jax.ShapeDtypeStruct((B, A), f32),
                   jax.ShapeDtypeStruct((B, 1), f32)),
    )(seg_sum, max_p, counts, cur_emb, action_mask,
      Wa1, ba1.reshape(1, 256), Wa2, ba2.reshape(1, A),
      Wc1, bc1.reshape(1, 256), Wc2, bc2.reshape(1, 1))


def kernel(node_features, edge_index, edge_features, action_mask, current_node,
           batch, W1, b1, W2, b2, W3, b3, Wa1, ba1, Wa2, ba2, Wc1, bc1,
           Wc2, bc2):
    del edge_index, edge_features
    return _run(node_features, action_mask, current_node, batch,
                W1, b1, W2, b2, W3, b3, Wa1, ba1, Wa2, ba2,
                Wc1, bc1, Wc2, bc2)


# speedup vs baseline: 1.2684x; 1.1619x over previous
import functools

import jax
import jax.numpy as jnp
from jax import lax
from jax.experimental import pallas as pl
from jax.experimental.pallas import tpu as pltpu
from jax.experimental.pallas import tpu_sc as plsc

N, F, E = 10000, 128, 320000
B, A, H = 256, 10, 128

NC, NS, L = 2, 16, 16
RPT = 320
NPAD = NC * NS * RPT
BACC = B + 16
CHUNK = 64
NCH = RPT // CHUNK
GPT = B // NS
CUR_PT = B // (NC * NS)


def _elu(x):
    return jnp.where(x > 0, x, jnp.exp(x) - 1.0)


def _encode_scan_kernel(nf_ref, w1_ref, b1_ref, w2_ref, b2_ref, w3_ref,
                        b3_ref, bcol_ref, mb_ref, brow_ref,
                        h_ref, maxp_ref, cnt_ref):
    f32 = jnp.float32
    x = nf_ref[...]
    h = _elu(jnp.dot(x, w1_ref[...], preferred_element_type=f32) + b1_ref[...])
    h = _elu(jnp.dot(h, w2_ref[...], preferred_element_type=f32) + b2_ref[...])
    h = _elu(jnp.dot(h, w3_ref[...], preferred_element_type=f32) + b3_ref[...])
    h_ref[pl.ds(0, N), :] = h
    h_ref[pl.ds(N, NPAD - N), :] = jnp.zeros((NPAD - N, F), dtype=f32)

    h_scan = h
    b_col = bcol_ref[...]
    step = 1
    while step < N:
        shifted = jnp.concatenate(
            [jnp.full((step, F), -jnp.inf, dtype=f32),
             h_scan[:N - step, :]], axis=0)
        bshift = jnp.concatenate(
            [jnp.full((step, 1), -1, dtype=jnp.int32),
             b_col[:N - step, :]], axis=0)
        pen = jnp.where(b_col == bshift, 0.0, -jnp.inf).astype(f32)
        h_scan = jnp.maximum(h_scan, shifted + pen)
        step *= 2

    C = 2000
    seg_iota = lax.broadcasted_iota(jnp.int32, (B, C), 0)
    max_p = jnp.zeros((B, F), dtype=f32)
    counts = jnp.zeros((B, 1), dtype=f32)
    for off in range(0, N, C):
        lastsel = (seg_iota == mb_ref[:, off:off + C]).astype(f32)
        max_p = max_p + jnp.dot(lastsel, h_scan[off:off + C, :],
                                preferred_element_type=f32)
        onehot = (seg_iota == brow_ref[:, off:off + C]).astype(f32)
        counts = counts + jnp.sum(onehot, axis=1, keepdims=True)

    maxp_ref[...] = jnp.where(counts > 0, max_p, -jnp.inf)
    cnt_ref[...] = counts


def _sc_pool_body(h_hbm, batch3d_hbm, cur_hbm,
                  out_sum, out_cur,
                  rows_v, idx_v, cur_idx, cur_rows, mbuf,
                  shared_sum, sem, sem2):
    c = lax.axis_index("c")
    s = lax.axis_index("s")
    wid = c * NS + s
    row0 = wid * RPT

    zero16 = jnp.zeros((L,), dtype=jnp.float32)

    rows_cp = pltpu.make_async_copy(h_hbm.at[pl.ds(row0, RPT)], rows_v, sem)
    rows_cp.start()
    idx_cp = pltpu.make_async_copy(batch3d_hbm.at[wid], idx_v, sem2)
    idx_cp.start()

    def init_z(i, carry):
        for j in range(F // L):
            mbuf[i, pl.ds(L * j, L)] = zero16
        return carry
    lax.fori_loop(0, GPT, init_z, 0)

    pltpu.sync_copy(mbuf, shared_sum.at[pl.ds(GPT * s, GPT)])

    @pl.when(s == 0)
    def _zero_pad_rows():
        pltpu.sync_copy(mbuf, shared_sum.at[pl.ds(B, BACC - B)])

    rows_cp.wait()
    idx_cp.wait()
    plsc.subcore_barrier()

    descs = []
    for k in range(NCH):
        descs.append(pltpu.async_copy(
            rows_v.at[pl.ds(CHUNK * k, CHUNK)],
            shared_sum.at[idx_v.at[k]], sem, add=True))

    pltpu.sync_copy(cur_hbm.at[pl.ds(wid * CUR_PT, CUR_PT)], cur_idx)
    pltpu.async_copy(h_hbm.at[cur_idx], cur_rows, sem2).wait()
    pltpu.sync_copy(cur_rows, out_cur.at[pl.ds(wid * CUR_PT, CUR_PT)])

    for d in descs:
        d.wait()

    plsc.subcore_barrier()

    pltpu.sync_copy(shared_sum.at[pl.ds(GPT * s, GPT)],
                    out_sum.at[c, pl.ds(GPT * s, GPT)])


@functools.lru_cache(maxsize=1)
def _make_sc_pool():
    return functools.partial(
        pl.kernel,
        out_type=(
            jax.ShapeDtypeStruct((NC, B, F), jnp.float32),
            jax.ShapeDtypeStruct((B, F), jnp.float32),
        ),
        mesh=plsc.VectorSubcoreMesh(core_axis_name="c", subcore_axis_name="s",
                                    num_cores=NC, num_subcores=NS),
        scratch_types=[
            pltpu.VMEM((RPT, F), jnp.float32),
            pltpu.VMEM((NCH, CHUNK), jnp.int32),
            pltpu.VMEM((CUR_PT,), jnp.int32),
            pltpu.VMEM((CUR_PT, F), jnp.float32),
            pltpu.VMEM((GPT, F), jnp.float32),
            pltpu.VMEM_SHARED((BACC, F), jnp.float32),
            pltpu.SemaphoreType.DMA,
            pltpu.SemaphoreType.DMA,
        ],
    )(_sc_pool_body)


def _heads_kernel(sum_ref, maxp_ref, cnt_ref, cur_ref, mask_ref,
                  wa1_ref, ba1_ref, wa2_ref, ba2_ref,
                  wc1_ref, bc1_ref, wc2_ref, bc2_ref,
                  probs_ref, values_ref):
    f32 = jnp.float32
    seg_sum = sum_ref[0] + sum_ref[1]
    counts = cnt_ref[...]
    mean_p = seg_sum / jnp.maximum(counts, 1.0)

    graph_emb = jnp.concatenate([mean_p, maxp_ref[...]], axis=-1)
    actor_in = jnp.concatenate([graph_emb, cur_ref[...]], axis=-1)
    a = _elu(jnp.dot(actor_in, wa1_ref[...], preferred_element_type=f32)
             + ba1_ref[...])
    logits = jnp.dot(a, wa2_ref[...], preferred_element_type=f32) + ba2_ref[...]

    amask = mask_ref[...]
    has_valid = jnp.sum(amask, axis=-1, keepdims=True) > 0
    safe_mask = jnp.where(has_valid, amask, jnp.ones_like(amask))
    logits = jnp.where(safe_mask == 0, -jnp.inf, logits)
    m = jnp.max(logits, axis=-1, keepdims=True)
    e = jnp.exp(logits - m)
    probs = e / jnp.sum(e, axis=-1, keepdims=True)
    nan_mask = jnp.any(jnp.isnan(probs), axis=-1, keepdims=True)
    probs_ref[...] = jnp.where(nan_mask, jnp.full_like(probs, 1.0 / A), probs)

    c = _elu(jnp.dot(graph_emb, wc1_ref[...], preferred_element_type=f32)
             + bc1_ref[...])
    values_ref[...] = (jnp.dot(c, wc2_ref[...], preferred_element_type=f32)
                       + bc2_ref[...])


@jax.jit
def _run(node_features, action_mask, current_node, batch,
         W1, b1, W2, b2, W3, b3, Wa1, ba1, Wa2, ba2, Wc1, bc1, Wc2, bc2):
    f32 = jnp.float32
    batch_i32 = batch.astype(jnp.int32)
    nxt = jnp.concatenate(
        [batch_i32[1:], jnp.full((1,), -1, dtype=jnp.int32)])
    mb = jnp.where(batch_i32 != nxt, batch_i32, -1)
    batch_pad = jnp.full((NPAD,), B, dtype=jnp.int32).at[:N].set(batch_i32)
    batch3d = batch_pad.reshape(NC * NS, NCH, CHUNK)
    cur = current_node.astype(jnp.int32)

    h, max_p, counts = pl.pallas_call(
        _encode_scan_kernel,
        out_shape=(jax.ShapeDtypeStruct((NPAD, F), f32),
                   jax.ShapeDtypeStruct((B, F), f32),
                   jax.ShapeDtypeStruct((B, 1), f32)),
    )(node_features, W1, b1.reshape(1, H), W2, b2.reshape(1, H),
      W3, b3.reshape(1, H), batch_i32.reshape(N, 1), mb.reshape(1, N),
      batch_i32.reshape(1, N))

    seg_sum, cur_emb = _make_sc_pool()(h, batch3d, cur)

    return pl.pallas_call(
        _heads_kernel,
        out_shape=(jax.ShapeDtypeStruct((B, A), f32),
                   jax.ShapeDtypeStruct((B, 1), f32)),
    )(seg_sum, max_p, counts, cur_emb, action_mask,
      Wa1, ba1.reshape(1, 256), Wa2, ba2.reshape(1, A),
      Wc1, bc1.reshape(1, 256), Wc2, bc2.reshape(1, 1))


def kernel(node_features, edge_index, edge_features, action_mask, current_node,
           batch, W1, b1, W2, b2, W3, b3, Wa1, ba1, Wa2, ba2, Wc1, bc1,
           Wc2, bc2):
    del edge_index, edge_features
    return _run(node_features, action_mask, current_node, batch,
                W1, b1, W2, b2, W3, b3, Wa1, ba1, Wa2, ba2,
                Wc1, bc1, Wc2, bc2)
